# baseline (speedup 1.0000x reference)
import functools

import jax
import jax.numpy as jnp
from jax.experimental import pallas as pl
from jax.experimental.pallas import tpu as pltpu

_F32 = jnp.float32


def _ln(x, g, b):
    mu = jnp.mean(x, axis=-1, keepdims=True)
    var = jnp.mean((x - mu) ** 2, axis=-1, keepdims=True)
    return (x - mu) * jax.lax.rsqrt(var + 1e-5) * g + b


def _knn_combine(q, xt, a, f2):
    qx, qy, qz = q[:, 0:1], q[:, 1:2], q[:, 2:3]
    xx, xy, xz = xt[0:1, :], xt[1:2, :], xt[2:3, :]
    dx, dy, dz = qx - xx, qy - xy, qz - xz
    d2 = dx * dx + dy * dy + dz * dz

    big = _F32(1e30)
    v1 = jnp.min(d2, axis=-1, keepdims=True)
    eq1 = d2 == v1
    dm = jnp.where(eq1, big, d2)
    v2 = jnp.min(dm, axis=-1, keepdims=True)
    eq2 = dm == v2
    dm2 = jnp.where(eq2, big, dm)
    v3 = jnp.min(dm2, axis=-1, keepdims=True)
    eq3 = dm2 == v3

    def wgt(m):
        return 1.0 / (jnp.sqrt(jnp.maximum(m, 1e-10)) + 1e-8)

    wa, wb, wc = wgt(v1), wgt(v2), wgt(v3)
    inv = 1.0 / (wa + wb + wc)
    zero = _F32(0.0)
    onehot = jnp.where(
        eq1, wa * inv,
        jnp.where(eq2, wb * inv, jnp.where(eq3, wc * inv, zero)))
    interp = jnp.dot(onehot, f2, preferred_element_type=_F32)
    return a + interp


def _mm(x, w, b):
    return jnp.dot(x, w, preferred_element_type=_F32) + b


def _k1_kernel(x0_ref, q0_ref, s0_ref, fp_ref,
               g10_ref, b10_ref, w10_ref, bb10_ref,
               g20_ref, bt20_ref, w20_ref, bb20_ref,
               x1_ref, q1_ref, s1_ref,
               g11_ref, b11_ref, w11_ref, bb11_ref,
               g21_ref, bt21_ref, w21_ref, bb21_ref,
               x2_ref, q2_ref, s2_ref,
               g12_ref, b12_ref, w12_ref, bb12_ref,
               g22_ref, bt22_ref, w22_ref, bb22_ref,
               out_ref, f2c_buf):
    i = pl.program_id(0)

    @pl.when(i == 0)
    def _():
        f2a = _mm(_ln(fp_ref[...], g20_ref[...], bt20_ref[...]),
                  w20_ref[...], bb20_ref[...])
        a0 = _mm(_ln(s0_ref[...], g10_ref[...], b10_ref[...]),
                 w10_ref[...], bb10_ref[...])
        f1 = _knn_combine(q0_ref[...], x0_ref[...], a0, f2a)
        f2b = _mm(_ln(f1, g21_ref[...], bt21_ref[...]),
                  w21_ref[...], bb21_ref[...])
        a1 = _mm(_ln(s1_ref[...], g11_ref[...], b11_ref[...]),
                 w11_ref[...], bb11_ref[...])
        f2 = _knn_combine(q1_ref[...], x1_ref[...], a1, f2b)
        f2c_buf[...] = _mm(_ln(f2, g22_ref[...], bt22_ref[...]),
                           w22_ref[...], bb22_ref[...])

    a2 = _mm(_ln(s2_ref[...], g12_ref[...], b12_ref[...]),
             w12_ref[...], bb12_ref[...])
    out_ref[...] = _knn_combine(q2_ref[...], x2_ref[...], a2, f2c_buf[...])


def _k2_kernel(x3_ref, q3_ref, s3_ref, fp_ref,
               g13_ref, b13_ref, w13_ref, bb13_ref,
               g23_ref, bt23_ref, w23_ref, bb23_ref,
               cw1_ref, cb1_ref, cg_ref, cbb_ref, cw2_ref, cb2_ref,
               out_ref, f2_buf, f4_buf, *, tile, nq, n):
    i = pl.program_id(0)

    @pl.when(i == 0)
    def _():
        f2_buf[...] = _mm(_ln(fp_ref[...], g23_ref[...], bt23_ref[...]),
                          w23_ref[...], bb23_ref[...])

    a3 = _mm(_ln(s3_ref[...], g13_ref[...], b13_ref[...]),
             w13_ref[...], bb13_ref[...])
    f4_buf[pl.ds(i * tile, tile), :] = _knn_combine(
        q3_ref[...], x3_ref[...], a3, f2_buf[...])

    @pl.when(i == nq - 1)
    def _():
        f = f4_buf[...]
        h = _mm(f, cw1_ref[...], cb1_ref[...])
        rows = jax.lax.broadcasted_iota(jnp.int32, h.shape, 0)
        mask = (rows < n).astype(_F32)
        invn = _F32(1.0 / n)
        mu = jnp.sum(h * mask, axis=0, keepdims=True) * invn
        var = jnp.sum(((h - mu) ** 2) * mask, axis=0, keepdims=True) * invn
        hn = (h - mu) * jax.lax.rsqrt(var + 1e-5) * cg_ref[...] + cbb_ref[...]
        hn = jnp.maximum(hn, 0.0)
        out_ref[...] = _mm(hn, cw2_ref[...], cb2_ref[...])


def _padq(sxyz, spad):
    return jnp.pad(sxyz, ((0, spad - sxyz.shape[0]), (0, 5)))


def _padx(xyz, mpad):
    return jnp.pad(xyz, ((0, mpad - xyz.shape[0]), (0, 5)),
                   constant_values=1e6).T


def _vec(v):
    return v.reshape(1, -1)


def kernel(feats, xyz0, sxyz0, sfeats0, xyz1, sxyz1, sfeats1,
           xyz2, sxyz2, sfeats2, xyz3, sxyz3, sfeats3,
           u0_ln1_g, u0_ln1_b, u0_w1, u0_b1, u0_ln2_g, u0_ln2_b, u0_w2, u0_b2,
           u1_ln1_g, u1_ln1_b, u1_w1, u1_b1, u1_ln2_g, u1_ln2_b, u1_w2, u1_b2,
           u2_ln1_g, u2_ln1_b, u2_w1, u2_b1, u2_ln2_g, u2_ln2_b, u2_w2, u2_b2,
           u3_ln1_g, u3_ln1_b, u3_w1, u3_b1, u3_ln2_g, u3_ln2_b, u3_w2, u3_b2,
           conf_w1, conf_b1, conf_bn_g, conf_bn_b, conf_w2, conf_b2):
    t2 = 512
    full = lambda i: (0, 0)
    tiled = lambda i: (i, 0)

    f0 = jnp.pad(feats, ((0, 128 - 39), (0, 0)))
    k1_in = [
        _padx(xyz0, 128), _padq(sxyz0, 160),
        jnp.pad(sfeats0, ((0, 160 - 156), (0, 0))), f0,
        _vec(u0_ln1_g), _vec(u0_ln1_b), u0_w1, _vec(u0_b1),
        _vec(u0_ln2_g), _vec(u0_ln2_b), u0_w2, _vec(u0_b2),
        _padx(xyz1, 160), _padq(sxyz1, 640),
        jnp.pad(sfeats1, ((0, 640 - 625), (0, 0))),
        _vec(u1_ln1_g), _vec(u1_ln1_b), u1_w1, _vec(u1_b1),
        _vec(u1_ln2_g), _vec(u1_ln2_b), u1_w2, _vec(u1_b2),
        _padx(xyz2, 640), _padq(sxyz2, 2560),
        jnp.pad(sfeats2, ((0, 2560 - 2500), (0, 0))),
        _vec(u2_ln1_g), _vec(u2_ln1_b), u2_w1, _vec(u2_b1),
        _vec(u2_ln2_g), _vec(u2_ln2_b), u2_w2, _vec(u2_b2),
    ]
    k1_specs = [
        pl.BlockSpec((8, 128), full), pl.BlockSpec((160, 8), full),
        pl.BlockSpec((160, 512), full), pl.BlockSpec((128, 512), full),
        pl.BlockSpec((1, 512), full), pl.BlockSpec((1, 512), full),
        pl.BlockSpec((512, 512), full), pl.BlockSpec((1, 512), full),
        pl.BlockSpec((1, 512), full), pl.BlockSpec((1, 512), full),
        pl.BlockSpec((512, 512), full), pl.BlockSpec((1, 512), full),
        pl.BlockSpec((8, 160), full), pl.BlockSpec((640, 8), full),
        pl.BlockSpec((640, 512), full),
        pl.BlockSpec((1, 512), full), pl.BlockSpec((1, 512), full),
        pl.BlockSpec((512, 512), full), pl.BlockSpec((1, 512), full),
        pl.BlockSpec((1, 512), full), pl.BlockSpec((1, 512), full),
        pl.BlockSpec((512, 512), full), pl.BlockSpec((1, 512), full),
        pl.BlockSpec((8, 640), full), pl.BlockSpec((t2, 8), tiled),
        pl.BlockSpec((t2, 256), tiled),
        pl.BlockSpec((1, 256), full), pl.BlockSpec((1, 256), full),
        pl.BlockSpec((256, 256), full), pl.BlockSpec((1, 256), full),
        pl.BlockSpec((1, 512), full), pl.BlockSpec((1, 512), full),
        pl.BlockSpec((512, 256), full), pl.BlockSpec((1, 256), full),
    ]
    f3 = pl.pallas_call(
        _k1_kernel,
        grid=(2560 // t2,),
        in_specs=k1_specs,
        out_specs=pl.BlockSpec((t2, 256), tiled),
        out_shape=jax.ShapeDtypeStruct((2560, 256), _F32),
        scratch_shapes=[pltpu.VMEM((640, 256), _F32)],
    )(*k1_in)

    t3 = 128
    spad3 = 10112
    nq = spad3 // t3
    n = sxyz3.shape[0]
    k2_in = [
        _padx(xyz3, 2560), _padq(sxyz3, spad3),
        jnp.pad(sfeats3, ((0, spad3 - n), (0, 0))), f3,
        _vec(u3_ln1_g), _vec(u3_ln1_b), u3_w1, _vec(u3_b1),
        _vec(u3_ln2_g), _vec(u3_ln2_b), u3_w2, _vec(u3_b2),
        conf_w1, _vec(conf_b1), _vec(conf_bn_g), _vec(conf_bn_b),
        conf_w2, _vec(conf_b2),
    ]
    k2_specs = [
        pl.BlockSpec((8, 2560), full), pl.BlockSpec((t3, 8), tiled),
        pl.BlockSpec((t3, 128), tiled), pl.BlockSpec((2560, 256), full),
        pl.BlockSpec((1, 128), full), pl.BlockSpec((1, 128), full),
        pl.BlockSpec((128, 128), full), pl.BlockSpec((1, 128), full),
        pl.BlockSpec((1, 256), full), pl.BlockSpec((1, 256), full),
        pl.BlockSpec((256, 128), full), pl.BlockSpec((1, 128), full),
        pl.BlockSpec((128, 128), full), pl.BlockSpec((1, 128), full),
        pl.BlockSpec((1, 128), full), pl.BlockSpec((1, 128), full),
        pl.BlockSpec((128, 1), full), pl.BlockSpec((1, 1), full),
    ]
    conf = pl.pallas_call(
        functools.partial(_k2_kernel, tile=t3, nq=nq, n=n),
        grid=(nq,),
        in_specs=k2_specs,
        out_specs=pl.BlockSpec((spad3, 1), full),
        out_shape=jax.ShapeDtypeStruct((spad3, 1), _F32),
        scratch_shapes=[pltpu.VMEM((2560, 128), _F32),
                        pltpu.VMEM((spad3, 128), _F32)],
    )(*k2_in)
    return conf[:n, :]


# speedup vs baseline: 1.1099x; 1.1099x over previous
import functools

import jax
import jax.numpy as jnp
from jax.experimental import pallas as pl
from jax.experimental.pallas import tpu as pltpu

_F32 = jnp.float32


def _ln(x, g, b):
    mu = jnp.mean(x, axis=-1, keepdims=True)
    var = jnp.mean((x - mu) ** 2, axis=-1, keepdims=True)
    return (x - mu) * jax.lax.rsqrt(var + 1e-5) * g + b


def _knn_combine(q, xt, a, f2):
    qx, qy, qz = q[:, 0:1], q[:, 1:2], q[:, 2:3]
    xx, xy, xz = xt[0:1, :], xt[1:2, :], xt[2:3, :]
    dx, dy, dz = qx - xx, qy - xy, qz - xz
    d2 = dx * dx + dy * dy + dz * dz

    big = _F32(1e30)
    v1 = jnp.min(d2, axis=-1, keepdims=True)
    eq1 = d2 == v1
    dm = jnp.where(eq1, big, d2)
    v2 = jnp.min(dm, axis=-1, keepdims=True)
    eq2 = dm == v2
    dm2 = jnp.where(eq2, big, dm)
    v3 = jnp.min(dm2, axis=-1, keepdims=True)
    eq3 = dm2 == v3

    def wgt(m):
        return 1.0 / (jnp.sqrt(jnp.maximum(m, 1e-10)) + 1e-8)

    wa, wb, wc = wgt(v1), wgt(v2), wgt(v3)
    inv = 1.0 / (wa + wb + wc)
    zero = _F32(0.0)
    onehot = jnp.where(
        eq1, wa * inv,
        jnp.where(eq2, wb * inv, jnp.where(eq3, wc * inv, zero)))
    interp = jnp.dot(onehot, f2, preferred_element_type=_F32)
    return a + interp


def _mm(x, w, b):
    return jnp.dot(x, w, preferred_element_type=_F32) + b


def _k1_kernel(x0_ref, q0_ref, s0_ref, fp_ref,
               g10_ref, b10_ref, w10_ref, bb10_ref,
               g20_ref, bt20_ref, w20_ref, bb20_ref,
               x1_ref, q1_ref, s1_ref,
               g11_ref, b11_ref, w11_ref, bb11_ref,
               g21_ref, bt21_ref, w21_ref, bb21_ref,
               x2_ref, q2_ref, s2_ref,
               g12_ref, b12_ref, w12_ref, bb12_ref,
               g22_ref, bt22_ref, w22_ref, bb22_ref,
               out_ref, f2c_buf):
    i = pl.program_id(0)

    @pl.when(i == 0)
    def _():
        f2a = _mm(_ln(fp_ref[...], g20_ref[...], bt20_ref[...]),
                  w20_ref[...], bb20_ref[...])
        a0 = _mm(_ln(s0_ref[...], g10_ref[...], b10_ref[...]),
                 w10_ref[...], bb10_ref[...])
        f1 = _knn_combine(q0_ref[...], x0_ref[...], a0, f2a)
        f2b = _mm(_ln(f1, g21_ref[...], bt21_ref[...]),
                  w21_ref[...], bb21_ref[...])
        a1 = _mm(_ln(s1_ref[...], g11_ref[...], b11_ref[...]),
                 w11_ref[...], bb11_ref[...])
        f2 = _knn_combine(q1_ref[...], x1_ref[...], a1, f2b)
        f2c_buf[...] = _mm(_ln(f2, g22_ref[...], bt22_ref[...]),
                           w22_ref[...], bb22_ref[...])

    a2 = _mm(_ln(s2_ref[...], g12_ref[...], b12_ref[...]),
             w12_ref[...], bb12_ref[...])
    out_ref[...] = _knn_combine(q2_ref[...], x2_ref[...], a2, f2c_buf[...])


def _k2_kernel(x3_ref, q3_ref, s3_ref, fp_ref,
               g13_ref, b13_ref, w13_ref, bb13_ref,
               g23_ref, bt23_ref, w23_ref, bb23_ref,
               cw1_ref, cb1_ref, cg_ref, cbb_ref, cw2_ref, cb2_ref,
               out_ref, f2_buf, f4_buf, *, tile, nq, n):
    i = pl.program_id(0)

    @pl.when(i == 0)
    def _():
        f2_buf[...] = _mm(_ln(fp_ref[...], g23_ref[...], bt23_ref[...]),
                          w23_ref[...], bb23_ref[...])

    a3 = _mm(_ln(s3_ref[...], g13_ref[...], b13_ref[...]),
             w13_ref[...], bb13_ref[...])
    f4_buf[pl.ds(i * tile, tile), :] = _knn_combine(
        q3_ref[...], x3_ref[...], a3, f2_buf[...])

    @pl.when(i == nq - 1)
    def _():
        f = f4_buf[...]
        h = _mm(f, cw1_ref[...], cb1_ref[...])
        rows = jax.lax.broadcasted_iota(jnp.int32, h.shape, 0)
        mask = (rows < n).astype(_F32)
        invn = _F32(1.0 / n)
        mu = jnp.sum(h * mask, axis=0, keepdims=True) * invn
        var = jnp.sum(((h - mu) ** 2) * mask, axis=0, keepdims=True) * invn
        hn = (h - mu) * jax.lax.rsqrt(var + 1e-5) * cg_ref[...] + cbb_ref[...]
        hn = jnp.maximum(hn, 0.0)
        out_ref[...] = _mm(hn, cw2_ref[...], cb2_ref[...])


def _padq(sxyz, spad):
    return jnp.pad(sxyz, ((0, spad - sxyz.shape[0]), (0, 5)))


def _padx(xyz, mpad):
    return jnp.pad(xyz, ((0, mpad - xyz.shape[0]), (0, 5)),
                   constant_values=1e6).T


def _vec(v):
    return v.reshape(1, -1)


def kernel(feats, xyz0, sxyz0, sfeats0, xyz1, sxyz1, sfeats1,
           xyz2, sxyz2, sfeats2, xyz3, sxyz3, sfeats3,
           u0_ln1_g, u0_ln1_b, u0_w1, u0_b1, u0_ln2_g, u0_ln2_b, u0_w2, u0_b2,
           u1_ln1_g, u1_ln1_b, u1_w1, u1_b1, u1_ln2_g, u1_ln2_b, u1_w2, u1_b2,
           u2_ln1_g, u2_ln1_b, u2_w1, u2_b1, u2_ln2_g, u2_ln2_b, u2_w2, u2_b2,
           u3_ln1_g, u3_ln1_b, u3_w1, u3_b1, u3_ln2_g, u3_ln2_b, u3_w2, u3_b2,
           conf_w1, conf_b1, conf_bn_g, conf_bn_b, conf_w2, conf_b2):
    t2 = 1280
    full = lambda i: (0, 0)
    tiled = lambda i: (i, 0)

    f0 = jnp.pad(feats, ((0, 128 - 39), (0, 0)))
    k1_in = [
        _padx(xyz0, 128), _padq(sxyz0, 160),
        jnp.pad(sfeats0, ((0, 160 - 156), (0, 0))), f0,
        _vec(u0_ln1_g), _vec(u0_ln1_b), u0_w1, _vec(u0_b1),
        _vec(u0_ln2_g), _vec(u0_ln2_b), u0_w2, _vec(u0_b2),
        _padx(xyz1, 160), _padq(sxyz1, 640),
        jnp.pad(sfeats1, ((0, 640 - 625), (0, 0))),
        _vec(u1_ln1_g), _vec(u1_ln1_b), u1_w1, _vec(u1_b1),
        _vec(u1_ln2_g), _vec(u1_ln2_b), u1_w2, _vec(u1_b2),
        _padx(xyz2, 640), _padq(sxyz2, 2560),
        jnp.pad(sfeats2, ((0, 2560 - 2500), (0, 0))),
        _vec(u2_ln1_g), _vec(u2_ln1_b), u2_w1, _vec(u2_b1),
        _vec(u2_ln2_g), _vec(u2_ln2_b), u2_w2, _vec(u2_b2),
    ]
    k1_specs = [
        pl.BlockSpec((8, 128), full), pl.BlockSpec((160, 8), full),
        pl.BlockSpec((160, 512), full), pl.BlockSpec((128, 512), full),
        pl.BlockSpec((1, 512), full), pl.BlockSpec((1, 512), full),
        pl.BlockSpec((512, 512), full), pl.BlockSpec((1, 512), full),
        pl.BlockSpec((1, 512), full), pl.BlockSpec((1, 512), full),
        pl.BlockSpec((512, 512), full), pl.BlockSpec((1, 512), full),
        pl.BlockSpec((8, 160), full), pl.BlockSpec((640, 8), full),
        pl.BlockSpec((640, 512), full),
        pl.BlockSpec((1, 512), full), pl.BlockSpec((1, 512), full),
        pl.BlockSpec((512, 512), full), pl.BlockSpec((1, 512), full),
        pl.BlockSpec((1, 512), full), pl.BlockSpec((1, 512), full),
        pl.BlockSpec((512, 512), full), pl.BlockSpec((1, 512), full),
        pl.BlockSpec((8, 640), full), pl.BlockSpec((t2, 8), tiled),
        pl.BlockSpec((t2, 256), tiled),
        pl.BlockSpec((1, 256), full), pl.BlockSpec((1, 256), full),
        pl.BlockSpec((256, 256), full), pl.BlockSpec((1, 256), full),
        pl.BlockSpec((1, 512), full), pl.BlockSpec((1, 512), full),
        pl.BlockSpec((512, 256), full), pl.BlockSpec((1, 256), full),
    ]
    f3 = pl.pallas_call(
        _k1_kernel,
        grid=(2560 // t2,),
        in_specs=k1_specs,
        out_specs=pl.BlockSpec((t2, 256), tiled),
        out_shape=jax.ShapeDtypeStruct((2560, 256), _F32),
        scratch_shapes=[pltpu.VMEM((640, 256), _F32)],
    )(*k1_in)

    t3 = 384
    spad3 = 10368
    nq = spad3 // t3
    n = sxyz3.shape[0]
    k2_in = [
        _padx(xyz3, 2560), _padq(sxyz3, spad3),
        jnp.pad(sfeats3, ((0, spad3 - n), (0, 0))), f3,
        _vec(u3_ln1_g), _vec(u3_ln1_b), u3_w1, _vec(u3_b1),
        _vec(u3_ln2_g), _vec(u3_ln2_b), u3_w2, _vec(u3_b2),
        conf_w1, _vec(conf_b1), _vec(conf_bn_g), _vec(conf_bn_b),
        conf_w2, _vec(conf_b2),
    ]
    k2_specs = [
        pl.BlockSpec((8, 2560), full), pl.BlockSpec((t3, 8), tiled),
        pl.BlockSpec((t3, 128), tiled), pl.BlockSpec((2560, 256), full),
        pl.BlockSpec((1, 128), full), pl.BlockSpec((1, 128), full),
        pl.BlockSpec((128, 128), full), pl.BlockSpec((1, 128), full),
        pl.BlockSpec((1, 256), full), pl.BlockSpec((1, 256), full),
        pl.BlockSpec((256, 128), full), pl.BlockSpec((1, 128), full),
        pl.BlockSpec((128, 128), full), pl.BlockSpec((1, 128), full),
        pl.BlockSpec((1, 128), full), pl.BlockSpec((1, 128), full),
        pl.BlockSpec((128, 1), full), pl.BlockSpec((1, 1), full),
    ]
    conf = pl.pallas_call(
        functools.partial(_k2_kernel, tile=t3, nq=nq, n=n),
        grid=(nq,),
        in_specs=k2_specs,
        out_specs=pl.BlockSpec((spad3, 1), full),
        out_shape=jax.ShapeDtypeStruct((spad3, 1), _F32),
        scratch_shapes=[pltpu.VMEM((2560, 128), _F32),
                        pltpu.VMEM((spad3, 128), _F32)],
    )(*k2_in)
    return conf[:n, :]


# speedup vs baseline: 1.1204x; 1.0094x over previous
import functools

import jax
import jax.numpy as jnp
from jax.experimental import pallas as pl
from jax.experimental.pallas import tpu as pltpu

_F32 = jnp.float32


def _ln(x, g, b):
    mu = jnp.mean(x, axis=-1, keepdims=True)
    var = jnp.mean((x - mu) ** 2, axis=-1, keepdims=True)
    return (x - mu) * jax.lax.rsqrt(var + 1e-5) * g + b


def _knn_combine(q, xt, a, f2):
    qx, qy, qz = q[:, 0:1], q[:, 1:2], q[:, 2:3]
    xx, xy, xz = xt[0:1, :], xt[1:2, :], xt[2:3, :]
    dx, dy, dz = qx - xx, qy - xy, qz - xz
    d2 = dx * dx + dy * dy + dz * dz

    big = _F32(1e30)
    v1 = jnp.min(d2, axis=-1, keepdims=True)
    eq1 = d2 == v1
    dm = jnp.where(eq1, big, d2)
    v2 = jnp.min(dm, axis=-1, keepdims=True)
    eq2 = dm == v2
    dm2 = jnp.where(eq2, big, dm)
    v3 = jnp.min(dm2, axis=-1, keepdims=True)
    eq3 = dm2 == v3

    def wgt(m):
        return 1.0 / (jnp.sqrt(jnp.maximum(m, 1e-10)) + 1e-8)

    wa, wb, wc = wgt(v1), wgt(v2), wgt(v3)
    inv = 1.0 / (wa + wb + wc)
    zero = _F32(0.0)
    onehot = jnp.where(
        eq1, wa * inv,
        jnp.where(eq2, wb * inv, jnp.where(eq3, wc * inv, zero)))
    interp = jnp.dot(onehot, f2, preferred_element_type=_F32)
    return a + interp


def _mm(x, w, b):
    return jnp.dot(x, w, preferred_element_type=_F32) + b


def _k1_kernel(x0_ref, q0_ref, s0_ref, fp_ref,
               g10_ref, b10_ref, w10_ref, bb10_ref,
               g20_ref, bt20_ref, w20_ref, bb20_ref,
               x1_ref, q1_ref, s1_ref,
               g11_ref, b11_ref, w11_ref, bb11_ref,
               g21_ref, bt21_ref, w21_ref, bb21_ref,
               x2_ref, q2_ref, s2_ref,
               g12_ref, b12_ref, w12_ref, bb12_ref,
               g22_ref, bt22_ref, w22_ref, bb22_ref,
               out_ref, f2c_buf):
    i = pl.program_id(0)

    @pl.when(i == 0)
    def _():
        f2a = _mm(_ln(fp_ref[...], g20_ref[...], bt20_ref[...]),
                  w20_ref[...], bb20_ref[...])
        a0 = _mm(_ln(s0_ref[...], g10_ref[...], b10_ref[...]),
                 w10_ref[...], bb10_ref[...])
        f1 = _knn_combine(q0_ref[...], x0_ref[...], a0, f2a)
        f2b = _mm(_ln(f1, g21_ref[...], bt21_ref[...]),
                  w21_ref[...], bb21_ref[...])
        a1 = _mm(_ln(s1_ref[...], g11_ref[...], b11_ref[...]),
                 w11_ref[...], bb11_ref[...])
        f2 = _knn_combine(q1_ref[...], x1_ref[...], a1, f2b)
        f2c_buf[...] = _mm(_ln(f2, g22_ref[...], bt22_ref[...]),
                           w22_ref[...], bb22_ref[...])

    a2 = _mm(_ln(s2_ref[...], g12_ref[...], b12_ref[...]),
             w12_ref[...], bb12_ref[...])
    out_ref[...] = _knn_combine(q2_ref[...], x2_ref[...], a2, f2c_buf[...])


def _k2_kernel(x3_ref, q3_ref, s3_ref, fp_ref,
               g13_ref, b13_ref, w13_ref, bb13_ref,
               g23_ref, bt23_ref, w23_ref, bb23_ref,
               cw1_ref, cb1_ref, cg_ref, cbb_ref, cw2_ref, cb2_ref,
               out_ref, f2_buf, f4_buf, *, tile, nq, n):
    i = pl.program_id(0)

    @pl.when(i == 0)
    def _():
        f2_buf[...] = _mm(_ln(fp_ref[...], g23_ref[...], bt23_ref[...]),
                          w23_ref[...], bb23_ref[...])

    a3 = _mm(_ln(s3_ref[...], g13_ref[...], b13_ref[...]),
             w13_ref[...], bb13_ref[...])
    f4_buf[pl.ds(i * tile, tile), :] = _knn_combine(
        q3_ref[...], x3_ref[...], a3, f2_buf[...])

    @pl.when(i == nq - 1)
    def _():
        f = f4_buf[...]
        h = _mm(f, cw1_ref[...], cb1_ref[...])
        rows = jax.lax.broadcasted_iota(jnp.int32, h.shape, 0)
        mask = (rows < n).astype(_F32)
        invn = _F32(1.0 / n)
        mu = jnp.sum(h * mask, axis=0, keepdims=True) * invn
        var = jnp.sum(((h - mu) ** 2) * mask, axis=0, keepdims=True) * invn
        hn = (h - mu) * jax.lax.rsqrt(var + 1e-5) * cg_ref[...] + cbb_ref[...]
        hn = jnp.maximum(hn, 0.0)
        out_ref[...] = _mm(hn, cw2_ref[...], cb2_ref[...])


def _padq(sxyz, spad):
    return jnp.pad(sxyz, ((0, spad - sxyz.shape[0]), (0, 5)))


def _padx(xyz, mpad):
    return jnp.pad(xyz, ((0, mpad - xyz.shape[0]), (0, 5)),
                   constant_values=1e6).T


def _vec(v):
    return v.reshape(1, -1)


def kernel(feats, xyz0, sxyz0, sfeats0, xyz1, sxyz1, sfeats1,
           xyz2, sxyz2, sfeats2, xyz3, sxyz3, sfeats3,
           u0_ln1_g, u0_ln1_b, u0_w1, u0_b1, u0_ln2_g, u0_ln2_b, u0_w2, u0_b2,
           u1_ln1_g, u1_ln1_b, u1_w1, u1_b1, u1_ln2_g, u1_ln2_b, u1_w2, u1_b2,
           u2_ln1_g, u2_ln1_b, u2_w1, u2_b1, u2_ln2_g, u2_ln2_b, u2_w2, u2_b2,
           u3_ln1_g, u3_ln1_b, u3_w1, u3_b1, u3_ln2_g, u3_ln2_b, u3_w2, u3_b2,
           conf_w1, conf_b1, conf_bn_g, conf_bn_b, conf_w2, conf_b2):
    t2 = 1280
    full = lambda i: (0, 0)
    tiled = lambda i: (i, 0)

    f0 = jnp.pad(feats, ((0, 128 - 39), (0, 0)))
    k1_in = [
        _padx(xyz0, 128), _padq(sxyz0, 160),
        jnp.pad(sfeats0, ((0, 160 - 156), (0, 0))), f0,
        _vec(u0_ln1_g), _vec(u0_ln1_b), u0_w1, _vec(u0_b1),
        _vec(u0_ln2_g), _vec(u0_ln2_b), u0_w2, _vec(u0_b2),
        _padx(xyz1, 160), _padq(sxyz1, 640),
        jnp.pad(sfeats1, ((0, 640 - 625), (0, 0))),
        _vec(u1_ln1_g), _vec(u1_ln1_b), u1_w1, _vec(u1_b1),
        _vec(u1_ln2_g), _vec(u1_ln2_b), u1_w2, _vec(u1_b2),
        _padx(xyz2, 640), _padq(sxyz2, 2560),
        jnp.pad(sfeats2, ((0, 2560 - 2500), (0, 0))),
        _vec(u2_ln1_g), _vec(u2_ln1_b), u2_w1, _vec(u2_b1),
        _vec(u2_ln2_g), _vec(u2_ln2_b), u2_w2, _vec(u2_b2),
    ]
    k1_specs = [
        pl.BlockSpec((8, 128), full), pl.BlockSpec((160, 8), full),
        pl.BlockSpec((160, 512), full), pl.BlockSpec((128, 512), full),
        pl.BlockSpec((1, 512), full), pl.BlockSpec((1, 512), full),
        pl.BlockSpec((512, 512), full), pl.BlockSpec((1, 512), full),
        pl.BlockSpec((1, 512), full), pl.BlockSpec((1, 512), full),
        pl.BlockSpec((512, 512), full), pl.BlockSpec((1, 512), full),
        pl.BlockSpec((8, 160), full), pl.BlockSpec((640, 8), full),
        pl.BlockSpec((640, 512), full),
        pl.BlockSpec((1, 512), full), pl.BlockSpec((1, 512), full),
        pl.BlockSpec((512, 512), full), pl.BlockSpec((1, 512), full),
        pl.BlockSpec((1, 512), full), pl.BlockSpec((1, 512), full),
        pl.BlockSpec((512, 512), full), pl.BlockSpec((1, 512), full),
        pl.BlockSpec((8, 640), full), pl.BlockSpec((t2, 8), tiled),
        pl.BlockSpec((t2, 256), tiled),
        pl.BlockSpec((1, 256), full), pl.BlockSpec((1, 256), full),
        pl.BlockSpec((256, 256), full), pl.BlockSpec((1, 256), full),
        pl.BlockSpec((1, 512), full), pl.BlockSpec((1, 512), full),
        pl.BlockSpec((512, 256), full), pl.BlockSpec((1, 256), full),
    ]
    f3 = pl.pallas_call(
        _k1_kernel,
        grid=(2560 // t2,),
        in_specs=k1_specs,
        out_specs=pl.BlockSpec((t2, 256), tiled),
        out_shape=jax.ShapeDtypeStruct((2560, 256), _F32),
        scratch_shapes=[pltpu.VMEM((640, 256), _F32)],
    )(*k1_in)

    t3 = 336
    spad3 = 10080
    nq = spad3 // t3
    n = sxyz3.shape[0]
    k2_in = [
        _padx(xyz3, 2560), _padq(sxyz3, spad3),
        jnp.pad(sfeats3, ((0, spad3 - n), (0, 0))), f3,
        _vec(u3_ln1_g), _vec(u3_ln1_b), u3_w1, _vec(u3_b1),
        _vec(u3_ln2_g), _vec(u3_ln2_b), u3_w2, _vec(u3_b2),
        conf_w1, _vec(conf_b1), _vec(conf_bn_g), _vec(conf_bn_b),
        conf_w2, _vec(conf_b2),
    ]
    k2_specs = [
        pl.BlockSpec((8, 2560), full), pl.BlockSpec((t3, 8), tiled),
        pl.BlockSpec((t3, 128), tiled), pl.BlockSpec((2560, 256), full),
        pl.BlockSpec((1, 128), full), pl.BlockSpec((1, 128), full),
        pl.BlockSpec((128, 128), full), pl.BlockSpec((1, 128), full),
        pl.BlockSpec((1, 256), full), pl.BlockSpec((1, 256), full),
        pl.BlockSpec((256, 128), full), pl.BlockSpec((1, 128), full),
        pl.BlockSpec((128, 128), full), pl.BlockSpec((1, 128), full),
        pl.BlockSpec((1, 128), full), pl.BlockSpec((1, 128), full),
        pl.BlockSpec((128, 1), full), pl.BlockSpec((1, 1), full),
    ]
    conf = pl.pallas_call(
        functools.partial(_k2_kernel, tile=t3, nq=nq, n=n),
        grid=(nq,),
        in_specs=k2_specs,
        out_specs=pl.BlockSpec((spad3, 1), full),
        out_shape=jax.ShapeDtypeStruct((spad3, 1), _F32),
        scratch_shapes=[pltpu.VMEM((2560, 128), _F32),
                        pltpu.VMEM((spad3, 128), _F32)],
    )(*k2_in)
    return conf[:n, :]


# speedup vs baseline: 1.1770x; 1.0505x over previous
import functools

import jax
import jax.numpy as jnp
from jax.experimental import pallas as pl
from jax.experimental.pallas import tpu as pltpu

_F32 = jnp.float32


def _ln(x, g, b):
    mu = jnp.mean(x, axis=-1, keepdims=True)
    var = jnp.mean((x - mu) ** 2, axis=-1, keepdims=True)
    return (x - mu) * jax.lax.rsqrt(var + 1e-5) * g + b


def _knn_combine(q, xt, a, f2):
    qx, qy, qz = q[:, 0:1], q[:, 1:2], q[:, 2:3]
    xx, xy, xz = xt[0:1, :], xt[1:2, :], xt[2:3, :]
    dx, dy, dz = qx - xx, qy - xy, qz - xz
    d2 = dx * dx + dy * dy + dz * dz

    big = _F32(1e30)
    v1 = jnp.min(d2, axis=-1, keepdims=True)
    eq1 = d2 == v1
    dm = jnp.where(eq1, big, d2)
    v2 = jnp.min(dm, axis=-1, keepdims=True)
    eq2 = dm == v2
    dm2 = jnp.where(eq2, big, dm)
    v3 = jnp.min(dm2, axis=-1, keepdims=True)
    eq3 = dm2 == v3

    def wgt(m):
        return 1.0 / (jnp.sqrt(jnp.maximum(m, 1e-10)) + 1e-8)

    wa, wb, wc = wgt(v1), wgt(v2), wgt(v3)
    inv = 1.0 / (wa + wb + wc)
    zero = _F32(0.0)
    onehot = jnp.where(
        eq1, wa * inv,
        jnp.where(eq2, wb * inv, jnp.where(eq3, wc * inv, zero)))
    interp = jnp.dot(onehot, f2, preferred_element_type=_F32)
    return a + interp


def _mm(x, w, b):
    return jnp.dot(x, w, preferred_element_type=_F32) + b


def _k1_kernel(x0_ref, q0_ref, s0_ref, fp_ref,
               g10_ref, b10_ref, w10_ref, bb10_ref,
               g20_ref, bt20_ref, w20_ref, bb20_ref,
               x1_ref, q1_ref, s1_ref,
               g11_ref, b11_ref, w11_ref, bb11_ref,
               g21_ref, bt21_ref, w21_ref, bb21_ref,
               x2_ref, q2_ref, s2_ref,
               g12_ref, b12_ref, w12_ref, bb12_ref,
               g22_ref, bt22_ref, w22_ref, bb22_ref,
               out_ref, f2c_buf):
    i = pl.program_id(0)

    @pl.when(i == 0)
    def _():
        f2a = _mm(_ln(fp_ref[...], g20_ref[...], bt20_ref[...]),
                  w20_ref[...], bb20_ref[...])
        a0 = _mm(_ln(s0_ref[...], g10_ref[...], b10_ref[...]),
                 w10_ref[...], bb10_ref[...])
        f1 = _knn_combine(q0_ref[...], x0_ref[...], a0, f2a)
        f2b = _mm(_ln(f1, g21_ref[...], bt21_ref[...]),
                  w21_ref[...], bb21_ref[...])
        a1 = _mm(_ln(s1_ref[...], g11_ref[...], b11_ref[...]),
                 w11_ref[...], bb11_ref[...])
        f2 = _knn_combine(q1_ref[...], x1_ref[...], a1, f2b)
        f2c_buf[...] = _mm(_ln(f2, g22_ref[...], bt22_ref[...]),
                           w22_ref[...], bb22_ref[...])

    a2 = _mm(_ln(s2_ref[...], g12_ref[...], b12_ref[...]),
             w12_ref[...], bb12_ref[...])
    out_ref[...] = _knn_combine(q2_ref[...], x2_ref[...], a2, f2c_buf[...])


def _k2_kernel(x3_ref, q3_ref, s3_ref, fp_ref,
               g13_ref, b13_ref, w13_ref, bb13_ref,
               g23_ref, bt23_ref, w23_ref, bb23_ref,
               cw1_ref, cb1_ref, cg_ref, cbb_ref, cw2_ref, cb2_ref,
               out_ref, f2_buf, f4_buf, *, tile, nq, n):
    i = pl.program_id(0)

    @pl.when(i == 0)
    def _():
        f2_buf[...] = _mm(_ln(fp_ref[...], g23_ref[...], bt23_ref[...]),
                          w23_ref[...], bb23_ref[...])

    a3 = _mm(_ln(s3_ref[...], g13_ref[...], b13_ref[...]),
             w13_ref[...], bb13_ref[...])
    f4_buf[pl.ds(i * tile, tile), :] = _knn_combine(
        q3_ref[...], x3_ref[...], a3, f2_buf[...])

    @pl.when(i == nq - 1)
    def _():
        f = f4_buf[...]
        h = _mm(f, cw1_ref[...], cb1_ref[...])
        rows = jax.lax.broadcasted_iota(jnp.int32, h.shape, 0)
        mask = (rows < n).astype(_F32)
        invn = _F32(1.0 / n)
        mu = jnp.sum(h * mask, axis=0, keepdims=True) * invn
        var = jnp.sum(((h - mu) ** 2) * mask, axis=0, keepdims=True) * invn
        hn = (h - mu) * jax.lax.rsqrt(var + 1e-5) * cg_ref[...] + cbb_ref[...]
        hn = jnp.maximum(hn, 0.0)
        out_ref[...] = _mm(hn, cw2_ref[...], cb2_ref[...])


def _padq(sxyz, spad):
    return jnp.pad(sxyz, ((0, spad - sxyz.shape[0]), (0, 5)))


def _padx(xyz, mpad):
    return jnp.pad(xyz, ((0, mpad - xyz.shape[0]), (0, 5)),
                   constant_values=1e6).T


def _vec(v):
    return v.reshape(1, -1)


def kernel(feats, xyz0, sxyz0, sfeats0, xyz1, sxyz1, sfeats1,
           xyz2, sxyz2, sfeats2, xyz3, sxyz3, sfeats3,
           u0_ln1_g, u0_ln1_b, u0_w1, u0_b1, u0_ln2_g, u0_ln2_b, u0_w2, u0_b2,
           u1_ln1_g, u1_ln1_b, u1_w1, u1_b1, u1_ln2_g, u1_ln2_b, u1_w2, u1_b2,
           u2_ln1_g, u2_ln1_b, u2_w1, u2_b1, u2_ln2_g, u2_ln2_b, u2_w2, u2_b2,
           u3_ln1_g, u3_ln1_b, u3_w1, u3_b1, u3_ln2_g, u3_ln2_b, u3_w2, u3_b2,
           conf_w1, conf_b1, conf_bn_g, conf_bn_b, conf_w2, conf_b2):
    t2 = 1280
    full = lambda i: (0, 0)
    tiled = lambda i: (i, 0)

    f0 = jnp.pad(feats, ((0, 128 - 39), (0, 0)))
    k1_in = [
        _padx(xyz0, 128), _padq(sxyz0, 160),
        jnp.pad(sfeats0, ((0, 160 - 156), (0, 0))), f0,
        _vec(u0_ln1_g), _vec(u0_ln1_b), u0_w1, _vec(u0_b1),
        _vec(u0_ln2_g), _vec(u0_ln2_b), u0_w2, _vec(u0_b2),
        _padx(xyz1, 160), _padq(sxyz1, 640),
        jnp.pad(sfeats1, ((0, 640 - 625), (0, 0))),
        _vec(u1_ln1_g), _vec(u1_ln1_b), u1_w1, _vec(u1_b1),
        _vec(u1_ln2_g), _vec(u1_ln2_b), u1_w2, _vec(u1_b2),
        _padx(xyz2, 640), _padq(sxyz2, 2560),
        jnp.pad(sfeats2, ((0, 2560 - 2500), (0, 0))),
        _vec(u2_ln1_g), _vec(u2_ln1_b), u2_w1, _vec(u2_b1),
        _vec(u2_ln2_g), _vec(u2_ln2_b), u2_w2, _vec(u2_b2),
    ]
    k1_specs = [
        pl.BlockSpec((8, 128), full), pl.BlockSpec((160, 8), full),
        pl.BlockSpec((160, 512), full), pl.BlockSpec((128, 512), full),
        pl.BlockSpec((1, 512), full), pl.BlockSpec((1, 512), full),
        pl.BlockSpec((512, 512), full), pl.BlockSpec((1, 512), full),
        pl.BlockSpec((1, 512), full), pl.BlockSpec((1, 512), full),
        pl.BlockSpec((512, 512), full), pl.BlockSpec((1, 512), full),
        pl.BlockSpec((8, 160), full), pl.BlockSpec((640, 8), full),
        pl.BlockSpec((640, 512), full),
        pl.BlockSpec((1, 512), full), pl.BlockSpec((1, 512), full),
        pl.BlockSpec((512, 512), full), pl.BlockSpec((1, 512), full),
        pl.BlockSpec((1, 512), full), pl.BlockSpec((1, 512), full),
        pl.BlockSpec((512, 512), full), pl.BlockSpec((1, 512), full),
        pl.BlockSpec((8, 640), full), pl.BlockSpec((t2, 8), tiled),
        pl.BlockSpec((t2, 256), tiled),
        pl.BlockSpec((1, 256), full), pl.BlockSpec((1, 256), full),
        pl.BlockSpec((256, 256), full), pl.BlockSpec((1, 256), full),
        pl.BlockSpec((1, 512), full), pl.BlockSpec((1, 512), full),
        pl.BlockSpec((512, 256), full), pl.BlockSpec((1, 256), full),
    ]
    f3 = pl.pallas_call(
        _k1_kernel,
        grid=(2560 // t2,),
        in_specs=k1_specs,
        out_specs=pl.BlockSpec((t2, 256), tiled),
        out_shape=jax.ShapeDtypeStruct((2560, 256), _F32),
        scratch_shapes=[pltpu.VMEM((640, 256), _F32)],
    )(*k1_in)

    t3 = 400
    spad3 = 10000
    nq = spad3 // t3
    n = sxyz3.shape[0]
    k2_in = [
        _padx(xyz3, 2560), _padq(sxyz3, spad3),
        jnp.pad(sfeats3, ((0, spad3 - n), (0, 0))), f3,
        _vec(u3_ln1_g), _vec(u3_ln1_b), u3_w1, _vec(u3_b1),
        _vec(u3_ln2_g), _vec(u3_ln2_b), u3_w2, _vec(u3_b2),
        conf_w1, _vec(conf_b1), _vec(conf_bn_g), _vec(conf_bn_b),
        conf_w2, _vec(conf_b2),
    ]
    k2_specs = [
        pl.BlockSpec((8, 2560), full), pl.BlockSpec((t3, 8), tiled),
        pl.BlockSpec((t3, 128), tiled), pl.BlockSpec((2560, 256), full),
        pl.BlockSpec((1, 128), full), pl.BlockSpec((1, 128), full),
        pl.BlockSpec((128, 128), full), pl.BlockSpec((1, 128), full),
        pl.BlockSpec((1, 256), full), pl.BlockSpec((1, 256), full),
        pl.BlockSpec((256, 128), full), pl.BlockSpec((1, 128), full),
        pl.BlockSpec((128, 128), full), pl.BlockSpec((1, 128), full),
        pl.BlockSpec((1, 128), full), pl.BlockSpec((1, 128), full),
        pl.BlockSpec((128, 1), full), pl.BlockSpec((1, 1), full),
    ]
    conf = pl.pallas_call(
        functools.partial(_k2_kernel, tile=t3, nq=nq, n=n),
        grid=(nq,),
        in_specs=k2_specs,
        out_specs=pl.BlockSpec((spad3, 1), full),
        out_shape=jax.ShapeDtypeStruct((spad3, 1), _F32),
        scratch_shapes=[pltpu.VMEM((2560, 128), _F32),
                        pltpu.VMEM((spad3, 128), _F32)],
    )(*k2_in)
    return conf[:n, :]


# speedup vs baseline: 1.2063x; 1.0248x over previous
import functools

import jax
import jax.numpy as jnp
from jax.experimental import pallas as pl
from jax.experimental.pallas import tpu as pltpu

_F32 = jnp.float32


def _ln(x, g, b):
    mu = jnp.mean(x, axis=-1, keepdims=True)
    var = jnp.mean((x - mu) ** 2, axis=-1, keepdims=True)
    return (x - mu) * jax.lax.rsqrt(var + 1e-5) * g + b


def _knn_combine(q, xt, a, f2):
    qx, qy, qz = q[:, 0:1], q[:, 1:2], q[:, 2:3]
    xx, xy, xz = xt[0:1, :], xt[1:2, :], xt[2:3, :]
    dx, dy, dz = qx - xx, qy - xy, qz - xz
    d2 = dx * dx + dy * dy + dz * dz

    big = _F32(1e30)
    v1 = jnp.min(d2, axis=-1, keepdims=True)
    eq1 = d2 == v1
    dm = jnp.where(eq1, big, d2)
    v2 = jnp.min(dm, axis=-1, keepdims=True)
    eq2 = dm == v2
    dm2 = jnp.where(eq2, big, dm)
    v3 = jnp.min(dm2, axis=-1, keepdims=True)
    eq3 = dm2 == v3

    def wgt(m):
        return 1.0 / (jnp.sqrt(jnp.maximum(m, 1e-10)) + 1e-8)

    wa, wb, wc = wgt(v1), wgt(v2), wgt(v3)
    inv = 1.0 / (wa + wb + wc)
    zero = _F32(0.0)
    onehot = jnp.where(
        eq1, wa * inv,
        jnp.where(eq2, wb * inv, jnp.where(eq3, wc * inv, zero)))
    interp = jnp.dot(onehot, f2, preferred_element_type=_F32)
    return a + interp


def _mm(x, w, b):
    return jnp.dot(x, w, preferred_element_type=_F32) + b


def _k1_kernel(x0_ref, q0_ref, s0_ref, fp_ref,
               g10_ref, b10_ref, w10_ref, bb10_ref,
               g20_ref, bt20_ref, w20_ref, bb20_ref,
               x1_ref, q1_ref, s1_ref,
               g11_ref, b11_ref, w11_ref, bb11_ref,
               g21_ref, bt21_ref, w21_ref, bb21_ref,
               x2_ref, q2_ref, s2_ref,
               g12_ref, b12_ref, w12_ref, bb12_ref,
               g22_ref, bt22_ref, w22_ref, bb22_ref,
               out_ref, f2c_buf):
    i = pl.program_id(0)

    @pl.when(i == 0)
    def _():
        f2a = _mm(_ln(fp_ref[...], g20_ref[...], bt20_ref[...]),
                  w20_ref[...], bb20_ref[...])
        a0 = _mm(_ln(s0_ref[...], g10_ref[...], b10_ref[...]),
                 w10_ref[...], bb10_ref[...])
        f1 = _knn_combine(q0_ref[...], x0_ref[...], a0, f2a)
        f2b = _mm(_ln(f1, g21_ref[...], bt21_ref[...]),
                  w21_ref[...], bb21_ref[...])
        a1 = _mm(_ln(s1_ref[...], g11_ref[...], b11_ref[...]),
                 w11_ref[...], bb11_ref[...])
        f2 = _knn_combine(q1_ref[...], x1_ref[...], a1, f2b)
        f2c_buf[...] = _mm(_ln(f2, g22_ref[...], bt22_ref[...]),
                           w22_ref[...], bb22_ref[...])

    a2 = _mm(_ln(s2_ref[...], g12_ref[...], b12_ref[...]),
             w12_ref[...], bb12_ref[...])
    out_ref[...] = _knn_combine(q2_ref[...], x2_ref[...], a2, f2c_buf[...])


def _k2_kernel(x3_ref, q3_ref, s3_ref, fp_ref,
               g13_ref, b13_ref, w13_ref, bb13_ref,
               g23_ref, bt23_ref, w23_ref, bb23_ref,
               cw1_ref, cb1_ref, cg_ref, cbb_ref, cw2_ref, cb2_ref,
               out_ref, f2_buf, f4_buf, *, tile, nq, n):
    i = pl.program_id(0)

    @pl.when(i == 0)
    def _():
        f2_buf[...] = _mm(_ln(fp_ref[...], g23_ref[...], bt23_ref[...]),
                          w23_ref[...], bb23_ref[...])

    a3 = _mm(_ln(s3_ref[...], g13_ref[...], b13_ref[...]),
             w13_ref[...], bb13_ref[...])
    f4_buf[pl.ds(i * tile, tile), :] = _knn_combine(
        q3_ref[...], x3_ref[...], a3, f2_buf[...])

    @pl.when(i == nq - 1)
    def _():
        f = f4_buf[...]
        h = _mm(f, cw1_ref[...], cb1_ref[...])
        rows = jax.lax.broadcasted_iota(jnp.int32, h.shape, 0)
        mask = (rows < n).astype(_F32)
        invn = _F32(1.0 / n)
        mu = jnp.sum(h * mask, axis=0, keepdims=True) * invn
        var = jnp.sum(((h - mu) ** 2) * mask, axis=0, keepdims=True) * invn
        hn = (h - mu) * jax.lax.rsqrt(var + 1e-5) * cg_ref[...] + cbb_ref[...]
        hn = jnp.maximum(hn, 0.0)
        out_ref[...] = _mm(hn, cw2_ref[...], cb2_ref[...])


def _padq(sxyz, spad):
    return jnp.pad(sxyz, ((0, spad - sxyz.shape[0]), (0, 5)))


def _padx(xyz, mpad):
    return jnp.pad(xyz, ((0, mpad - xyz.shape[0]), (0, 5)),
                   constant_values=1e6).T


def _vec(v):
    return v.reshape(1, -1)


def kernel(feats, xyz0, sxyz0, sfeats0, xyz1, sxyz1, sfeats1,
           xyz2, sxyz2, sfeats2, xyz3, sxyz3, sfeats3,
           u0_ln1_g, u0_ln1_b, u0_w1, u0_b1, u0_ln2_g, u0_ln2_b, u0_w2, u0_b2,
           u1_ln1_g, u1_ln1_b, u1_w1, u1_b1, u1_ln2_g, u1_ln2_b, u1_w2, u1_b2,
           u2_ln1_g, u2_ln1_b, u2_w1, u2_b1, u2_ln2_g, u2_ln2_b, u2_w2, u2_b2,
           u3_ln1_g, u3_ln1_b, u3_w1, u3_b1, u3_ln2_g, u3_ln2_b, u3_w2, u3_b2,
           conf_w1, conf_b1, conf_bn_g, conf_bn_b, conf_w2, conf_b2):
    t2 = 1280
    full = lambda i: (0, 0)
    tiled = lambda i: (i, 0)

    f0 = jnp.pad(feats, ((0, 128 - 39), (0, 0)))
    k1_in = [
        _padx(xyz0, 128), _padq(sxyz0, 160),
        jnp.pad(sfeats0, ((0, 160 - 156), (0, 0))), f0,
        _vec(u0_ln1_g), _vec(u0_ln1_b), u0_w1, _vec(u0_b1),
        _vec(u0_ln2_g), _vec(u0_ln2_b), u0_w2, _vec(u0_b2),
        _padx(xyz1, 160), _padq(sxyz1, 640),
        jnp.pad(sfeats1, ((0, 640 - 625), (0, 0))),
        _vec(u1_ln1_g), _vec(u1_ln1_b), u1_w1, _vec(u1_b1),
        _vec(u1_ln2_g), _vec(u1_ln2_b), u1_w2, _vec(u1_b2),
        _padx(xyz2, 640), _padq(sxyz2, 2560),
        jnp.pad(sfeats2, ((0, 2560 - 2500), (0, 0))),
        _vec(u2_ln1_g), _vec(u2_ln1_b), u2_w1, _vec(u2_b1),
        _vec(u2_ln2_g), _vec(u2_ln2_b), u2_w2, _vec(u2_b2),
    ]
    k1_specs = [
        pl.BlockSpec((8, 128), full), pl.BlockSpec((160, 8), full),
        pl.BlockSpec((160, 512), full), pl.BlockSpec((128, 512), full),
        pl.BlockSpec((1, 512), full), pl.BlockSpec((1, 512), full),
        pl.BlockSpec((512, 512), full), pl.BlockSpec((1, 512), full),
        pl.BlockSpec((1, 512), full), pl.BlockSpec((1, 512), full),
        pl.BlockSpec((512, 512), full), pl.BlockSpec((1, 512), full),
        pl.BlockSpec((8, 160), full), pl.BlockSpec((640, 8), full),
        pl.BlockSpec((640, 512), full),
        pl.BlockSpec((1, 512), full), pl.BlockSpec((1, 512), full),
        pl.BlockSpec((512, 512), full), pl.BlockSpec((1, 512), full),
        pl.BlockSpec((1, 512), full), pl.BlockSpec((1, 512), full),
        pl.BlockSpec((512, 512), full), pl.BlockSpec((1, 512), full),
        pl.BlockSpec((8, 640), full), pl.BlockSpec((t2, 8), tiled),
        pl.BlockSpec((t2, 256), tiled),
        pl.BlockSpec((1, 256), full), pl.BlockSpec((1, 256), full),
        pl.BlockSpec((256, 256), full), pl.BlockSpec((1, 256), full),
        pl.BlockSpec((1, 512), full), pl.BlockSpec((1, 512), full),
        pl.BlockSpec((512, 256), full), pl.BlockSpec((1, 256), full),
    ]
    f3 = pl.pallas_call(
        _k1_kernel,
        grid=(2560 // t2,),
        in_specs=k1_specs,
        out_specs=pl.BlockSpec((t2, 256), tiled),
        out_shape=jax.ShapeDtypeStruct((2560, 256), _F32),
        scratch_shapes=[pltpu.VMEM((640, 256), _F32)],
    )(*k1_in)

    t3 = 1000
    spad3 = 10000
    nq = spad3 // t3
    n = sxyz3.shape[0]
    k2_in = [
        _padx(xyz3, 2560), _padq(sxyz3, spad3),
        jnp.pad(sfeats3, ((0, spad3 - n), (0, 0))), f3,
        _vec(u3_ln1_g), _vec(u3_ln1_b), u3_w1, _vec(u3_b1),
        _vec(u3_ln2_g), _vec(u3_ln2_b), u3_w2, _vec(u3_b2),
        conf_w1, _vec(conf_b1), _vec(conf_bn_g), _vec(conf_bn_b),
        conf_w2, _vec(conf_b2),
    ]
    k2_specs = [
        pl.BlockSpec((8, 2560), full), pl.BlockSpec((t3, 8), tiled),
        pl.BlockSpec((t3, 128), tiled), pl.BlockSpec((2560, 256), full),
        pl.BlockSpec((1, 128), full), pl.BlockSpec((1, 128), full),
        pl.BlockSpec((128, 128), full), pl.BlockSpec((1, 128), full),
        pl.BlockSpec((1, 256), full), pl.BlockSpec((1, 256), full),
        pl.BlockSpec((256, 128), full), pl.BlockSpec((1, 128), full),
        pl.BlockSpec((128, 128), full), pl.BlockSpec((1, 128), full),
        pl.BlockSpec((1, 128), full), pl.BlockSpec((1, 128), full),
        pl.BlockSpec((128, 1), full), pl.BlockSpec((1, 1), full),
    ]
    conf = pl.pallas_call(
        functools.partial(_k2_kernel, tile=t3, nq=nq, n=n),
        grid=(nq,),
        in_specs=k2_specs,
        out_specs=pl.BlockSpec((spad3, 1), full),
        out_shape=jax.ShapeDtypeStruct((spad3, 1), _F32),
        scratch_shapes=[pltpu.VMEM((2560, 128), _F32),
                        pltpu.VMEM((spad3, 128), _F32)],
    )(*k2_in)
    return conf[:n, :]
